# baseline (speedup 1.0000x reference)
import functools

import jax
import jax.numpy as jnp
from jax import lax
from jax.experimental import pallas as pl
from jax.experimental.pallas import tpu as pltpu
from jax.experimental.pallas import tpu_sc as plsc

_N = 10000
_E = 320000
_D = 128
_R = 8

_NC = 2
_NS = 16
_NT = _NC * _NS
_NPASS = _D // _NT
_SEC = 3200
_NSECS = _E // _SEC
_VPS = _SEC // 16
_ZW = _R * _N

_BN = 400
_NB = _N // _BN


def _sidx_body(et_ref, dst_ref, o_ref):
    o_ref[...] = et_ref[...] * _N + dst_ref[...]


def _sidx(et2d, dst2d):
    rows = et2d.shape[0]
    return pl.pallas_call(
        _sidx_body,
        grid=(1,),
        in_specs=[
            pl.BlockSpec((rows, 128), lambda i: (0, 0)),
            pl.BlockSpec((rows, 128), lambda i: (0, 0)),
        ],
        out_specs=pl.BlockSpec((rows, 128), lambda i: (0, 0)),
        out_shape=jax.ShapeDtypeStruct((rows, 128), jnp.int32),
    )(et2d, dst2d)


def _tr_body(x_ref, o_ref):
    o_ref[...] = x_ref[...].T


def _transpose(x):
    return pl.pallas_call(
        _tr_body,
        grid=(1,),
        in_specs=[pl.BlockSpec((_N, _D), lambda i: (0, 0))],
        out_specs=pl.BlockSpec((_D, _N), lambda i: (0, 0)),
        out_shape=jax.ShapeDtypeStruct((_D, _N), jnp.float32),
    )(x)


def _make_sc_zagg():
    mesh = plsc.VectorSubcoreMesh(core_axis_name="c", subcore_axis_name="s")

    @functools.partial(
        pl.kernel,
        mesh=mesh,
        compiler_params=pltpu.CompilerParams(needs_layout_passes=False),
        out_type=jax.ShapeDtypeStruct((_NPASS * _NT * _ZW,), jnp.float32),
        scratch_types=[
            pltpu.VMEM((_N,), jnp.float32),
            pltpu.VMEM((_ZW,), jnp.float32),
            [pltpu.VMEM((_SEC,), jnp.int32)] * 2,
            [pltpu.VMEM((_SEC,), jnp.int32)] * 2,
            pltpu.SemaphoreType.DMA,
            pltpu.SemaphoreType.DMA,
            [pltpu.SemaphoreType.DMA] * 2,
            [pltpu.SemaphoreType.DMA] * 2,
        ],
    )
    def sc_zagg(xt_hbm, gsrc_hbm, sidx_hbm, zeros_hbm, out_hbm,
                xcol_v, z_v, gs_v, si_v, semx, semz, gsems, ssems):
        c = lax.axis_index("c")
        s = lax.axis_index("s")
        w = s * _NC + c

        def fetch_sec(k, b):
            pltpu.async_copy(gsrc_hbm.at[pl.ds(k * _SEC, _SEC)], gs_v[b],
                             gsems[b])
            pltpu.async_copy(sidx_hbm.at[pl.ds(k * _SEC, _SEC)], si_v[b],
                             ssems[b])

        def wait_sec(k, b):
            pltpu.make_async_copy(gsrc_hbm.at[pl.ds(k * _SEC, _SEC)],
                                  gs_v[b], gsems[b]).wait()
            pltpu.make_async_copy(sidx_hbm.at[pl.ds(k * _SEC, _SEC)],
                                  si_v[b], ssems[b]).wait()

        def do_pass(p, carry):
            col = p * _NT + w
            cx = pltpu.async_copy(xt_hbm.at[pl.ds(col * _N, _N)], xcol_v,
                                  semx)
            cz = pltpu.async_copy(zeros_hbm, z_v, semz)
            fetch_sec(0, 0)
            cx.wait()
            cz.wait()

            def run_sec(k, b):
                @pl.when(k + 1 < _NSECS)
                def _():
                    fetch_sec(k + 1, 1 - b)

                wait_sec(k, b)

                def vecs(i, carry3):
                    base = i * (16 * 8)
                    for u in range(8):
                        off = base + u * 16
                        sv = gs_v[b][pl.ds(off, 16)]
                        iv = si_v[b][pl.ds(off, 16)]
                        vals = plsc.load_gather(xcol_v, [sv])
                        plsc.addupdate_scatter(z_v, [iv], vals)
                    return carry3

                lax.fori_loop(0, _VPS // 8, vecs, 0)

            def section2(k2, carry2):
                run_sec(2 * k2, 0)
                run_sec(2 * k2 + 1, 1)
                return carry2

            lax.fori_loop(0, _NSECS // 2, section2, 0)
            for r_ in range(_R):
                pltpu.sync_copy(z_v.at[pl.ds(r_ * _N, _N)],
                                out_hbm.at[pl.ds(r_ * _D * _N + col * _N, _N)])
            return carry

        lax.fori_loop(0, _NPASS, do_pass, 0)

    return sc_zagg


_sc_zagg = _make_sc_zagg()


def _dotT(zblk, wblk):
    return lax.dot_general(zblk, wblk, (((0,), (0,)), ((), ())),
                           preferred_element_type=jnp.float32)


def _combine1_body(z_ref, x_ref, w_ref, b_ref, oh_ref, oht_ref, acc_ref):
    r = pl.program_id(0)

    @pl.when(r == 0)
    def _():
        acc_ref[...] = jnp.zeros_like(acc_ref)

    @pl.when(r < _R)
    def _():
        acc_ref[...] += _dotT(z_ref[0], w_ref[0])

    @pl.when(r == _R)
    def _():
        h = jnp.maximum(
            acc_ref[...]
            + jnp.dot(x_ref[...], w_ref[0], preferred_element_type=jnp.float32)
            + b_ref[...], 0.0)
        oh_ref[...] = h
        oht_ref[...] = h.T


def _combine1(z, x, wall, b):
    return pl.pallas_call(
        _combine1_body,
        grid=(_R + 1,),
        in_specs=[
            pl.BlockSpec((1, _D, _N), lambda r: (jnp.minimum(r, _R - 1), 0, 0)),
            pl.BlockSpec((_N, _D), lambda r: (0, 0)),
            pl.BlockSpec((1, _D, _D), lambda r: (r, 0, 0)),
            pl.BlockSpec((1, _D), lambda r: (0, 0)),
        ],
        out_specs=[
            pl.BlockSpec((_N, _D), lambda r: (0, 0)),
            pl.BlockSpec((_D, _N), lambda r: (0, 0)),
        ],
        out_shape=[
            jax.ShapeDtypeStruct((_N, _D), jnp.float32),
            jax.ShapeDtypeStruct((_D, _N), jnp.float32),
        ],
        scratch_shapes=[pltpu.VMEM((_N, _D), jnp.float32)],
    )(z, x, wall, b)


def _combine2_body(z_ref, x_ref, w_ref, b_ref, fcw_ref, fcb_ref, o_ref,
                   acc_ref):
    r = pl.program_id(0)

    @pl.when(r == 0)
    def _():
        acc_ref[...] = jnp.zeros_like(acc_ref)

    @pl.when(r < _R)
    def _():
        acc_ref[...] += _dotT(z_ref[0], w_ref[0])

    @pl.when(r == _R)
    def _():
        h = jnp.maximum(
            acc_ref[...]
            + jnp.dot(x_ref[...], w_ref[0], preferred_element_type=jnp.float32)
            + b_ref[...], 0.0)
        hg = jnp.sum(h, axis=0, keepdims=True) * (1.0 / _N)
        zz = jnp.sum(hg * fcw_ref[...], keepdims=True) + fcb_ref[...]
        o_ref[...] = 1.0 / (1.0 + jnp.exp(-zz))


def _combine2(z, x, wall, b, fcw_row, fcb):
    return pl.pallas_call(
        _combine2_body,
        grid=(_R + 1,),
        in_specs=[
            pl.BlockSpec((1, _D, _N), lambda r: (jnp.minimum(r, _R - 1), 0, 0)),
            pl.BlockSpec((_N, _D), lambda r: (0, 0)),
            pl.BlockSpec((1, _D, _D), lambda r: (r, 0, 0)),
            pl.BlockSpec((1, _D), lambda r: (0, 0)),
            pl.BlockSpec((1, _D), lambda r: (0, 0)),
            pl.BlockSpec((1, 1), lambda r: (0, 0)),
        ],
        out_specs=pl.BlockSpec((1, 1), lambda r: (0, 0)),
        out_shape=jax.ShapeDtypeStruct((1, 1), jnp.float32),
        scratch_shapes=[pltpu.VMEM((_N, _D), jnp.float32)],
    )(z, x, wall, b, fcw_row, fcb)


def kernel(in_feat, edge_index, e_types, W1, Wself1, b1, W2, Wself2, b2,
           fc_w, fc_b):
    src = edge_index[0]
    dst = edge_index[1]

    sidx = _sidx(e_types.reshape(-1, 128), dst.reshape(-1, 128)).reshape(_E)
    zeros = jnp.zeros((_ZW,), jnp.float32)
    wall1 = jnp.concatenate([W1, Wself1[None]], axis=0)
    wall2 = jnp.concatenate([W2, Wself2[None]], axis=0)

    xt0 = _transpose(in_feat).reshape(_D * _N)
    z1 = _sc_zagg(xt0, src, sidx, zeros).reshape(_R, _D, _N)
    h1, h1t = _combine1(z1, in_feat, wall1, b1.reshape(1, _D))
    z2 = _sc_zagg(h1t.reshape(_D * _N), src, sidx, zeros).reshape(_R, _D, _N)
    return _combine2(z2, h1, wall2, b2.reshape(1, _D), fc_w.reshape(1, _D),
                     fc_b.reshape(1, 1))
